# baseline (speedup 1.0000x reference)
import functools

import jax
import jax.numpy as jnp
from jax import lax
from jax.experimental import pallas as pl
from jax.experimental.pallas import tpu as pltpu
from jax.experimental.pallas import tpu_sc as plsc

N = 10000
NPAD = 10240
R = 3
E = 200000
M = 200000
B = 128
CH = 128
D = 64
NM = 13
DUMP = 10016

EPAD = 200064
ROWS_B = 4696
NSB_B = ROWS_B // 8

EPAD_M = 200704
ROWS_M1 = EPAD_M // B
NSB_M1 = ROWS_M1 // 8

_f32 = jnp.float32
_i32 = jnp.int32


def _xt_body(x_ref, w_ref, o_ref):
    o_ref[0] = jnp.dot(x_ref[...], w_ref[0], preferred_element_type=_f32)


def _compute_xt(x_pad, w4):
    return pl.pallas_call(
        _xt_body,
        grid=(4, 10),
        in_specs=[
            pl.BlockSpec((NPAD // 10, CH), lambda i, j: (j, 0)),
            pl.BlockSpec((1, CH, CH), lambda i, j: (i, 0, 0)),
        ],
        out_specs=pl.BlockSpec((1, NPAD // 10, CH), lambda i, j: (i, j, 0)),
        out_shape=jax.ShapeDtypeStruct((4, NPAD, CH), _f32),
    )(x_pad, w4)


def _fill_zero_2d(buf, rows):
    def body(e, _):
        for j in range(CH // 16):
            buf[e, pl.ds(16 * j, 16)] = jnp.zeros((16,), _f32)
        return 0
    lax.fori_loop(0, rows, body, 0)


def _fill_zero_1d(buf, n):
    def body(g, _):
        buf[pl.ds(16 * g, 16)] = jnp.zeros((16,), _f32)
        return 0
    lax.fori_loop(0, n // 16, body, 0)


def _scale_chunk(rows_ref, val8, j):
    def body(e, _):
        e_hi = (e // 16) * 16
        vals16 = val8[j, pl.ds(e_hi, 16)]
        w16 = vals16.at[jnp.full((16,), 0, _i32) + (e - e_hi)].get(
            mode="promise_in_bounds")
        for c in range(CH // 16):
            sl = pl.ds(16 * c, 16)
            rows_ref[e, sl] = rows_ref[e, sl] * w16
        return 0
    lax.fori_loop(0, B, body, 0, unroll=8)


def _process_superblock(table, col2, row2, val2, sbk,
                        col8, row8, val8, rb0, rb1,
                        acc, sem_i, sem_g0, sem_g1, sem_s0, sem_s1,
                        degs=None, onesv=None):
    i1 = pltpu.async_copy(col2.at[pl.ds(8 * sbk, 8)], col8, sem_i)
    i2 = pltpu.async_copy(row2.at[pl.ds(8 * sbk, 8)], row8, sem_i)
    i3 = pltpu.async_copy(val2.at[pl.ds(8 * sbk, 8)], val8, sem_i)
    i1.wait()
    i2.wait()
    i3.wait()
    rbs = (rb0, rb1)
    sgs = (sem_g0, sem_g1)
    gat = [None] * 8
    gat[0] = pltpu.async_copy(table.at[col8.at[0]], rbs[0], sgs[0])
    for j in range(8):
        b = j % 2
        if j < 7:
            gat[j + 1] = pltpu.async_copy(
                table.at[col8.at[j + 1]], rbs[1 - b], sgs[1 - b])
        gat[j].wait()
        _scale_chunk(rbs[b], val8, j)
        pltpu.sync_copy(rbs[b], acc.at[row8.at[j]], add=True)
        if degs is not None:
            pltpu.sync_copy(onesv, degs.at[row8.at[j]], add=True)


def _relation_kernel(xt_hbm, col2, row2, val2, hp_out, deg_out,
                     col8, row8, val8, rb0, rb1, onesv,
                     acc, degs, sem_i, sem_g0, sem_g1, sem_s0, sem_s1):
    c = lax.axis_index("c")
    s = lax.axis_index("s")
    w = s * 2 + c

    _fill_zero_2d(rb0, B)
    for t in range(5):
        pltpu.sync_copy(rb0, acc.at[pl.ds(640 * s + B * t, B)])
    _fill_zero_1d(onesv, B)
    for t in range(5):
        pltpu.sync_copy(onesv, degs.at[pl.ds(640 * s + B * t, B)])
    for g in range(B // 16):
        onesv[pl.ds(16 * g, 16)] = jnp.ones((16,), _f32)
    plsc.subcore_barrier()

    def body(i, _):
        sbk = w + 32 * i

        @pl.when(sbk < NSB_B)
        def _():
            _process_superblock(xt_hbm, col2, row2, val2, sbk,
                                col8, row8, val8, rb0, rb1,
                                acc, sem_i, sem_g0, sem_g1, sem_s0, sem_s1,
                                degs=degs, onesv=onesv)
        return 0
    lax.fori_loop(0, (NSB_B + 31) // 32, body, 0)

    plsc.subcore_barrier()
    pltpu.sync_copy(acc.at[pl.ds(640 * s, 640)],
                    hp_out.at[c, pl.ds(640 * s, 640)])
    pltpu.sync_copy(degs.at[pl.ds(640 * s, 640)],
                    deg_out.at[pl.ds(c * NPAD + 640 * s, 640)])


def _relation_aggregate(xt_flat, col2, row2, val2):
    mesh = plsc.VectorSubcoreMesh(core_axis_name="c", subcore_axis_name="s")
    f = functools.partial(
        pl.kernel,
        out_type=[
            jax.ShapeDtypeStruct((2, NPAD, CH), _f32),
            jax.ShapeDtypeStruct((2 * NPAD,), _f32),
        ],
        mesh=mesh,
        scratch_types=[
            pltpu.VMEM((8, B), _i32),
            pltpu.VMEM((8, B), _i32),
            pltpu.VMEM((8, B), _f32),
            pltpu.VMEM((B, CH), _f32),
            pltpu.VMEM((B, CH), _f32),
            pltpu.VMEM((B,), _f32),
            pltpu.VMEM_SHARED((NPAD, CH), _f32),
            pltpu.VMEM_SHARED((NPAD,), _f32),
            pltpu.SemaphoreType.DMA,
            pltpu.SemaphoreType.DMA,
            pltpu.SemaphoreType.DMA,
            pltpu.SemaphoreType.DMA,
            pltpu.SemaphoreType.DMA,
        ],
    )(_relation_kernel)
    return f(xt_flat, col2, row2, val2)


def _combine_body(hp_ref, deg_ref, xt4_ref, b_ref, o_ref):
    i = pl.program_id(0)
    bn = NPAD // 10
    deg = (deg_ref[pl.ds(i * bn, bn)]
           + deg_ref[pl.ds(NPAD + i * bn, bn)])
    norm = jnp.where(deg > 0, 1.0 / jnp.maximum(deg, 1.0), 0.0)
    h = (hp_ref[0] + hp_ref[1]) * norm[:, None] + xt4_ref[0] + b_ref[...]
    o_ref[...] = h


def _combine(hp, degp, xt4, bias):
    bn = NPAD // 10
    return pl.pallas_call(
        _combine_body,
        grid=(10,),
        in_specs=[
            pl.BlockSpec((2, bn, CH), lambda i: (0, i, 0)),
            pl.BlockSpec((2 * NPAD,), lambda i: (0,)),
            pl.BlockSpec((1, bn, CH), lambda i: (3, i, 0)),
            pl.BlockSpec((CH,), lambda i: (0,)),
        ],
        out_specs=pl.BlockSpec((bn, CH), lambda i: (i, 0)),
        out_shape=jax.ShapeDtypeStruct((NPAD, CH), _f32),
    )(hp, degp, xt4, bias)


def _motif_kernel(h_hbm, col2, row2, val2, mot_out,
                  col8, row8, val8, rb0, rb1,
                  acc, sem_i, sem_g0, sem_g1, sem_s0, sem_s1):
    c = lax.axis_index("c")
    s = lax.axis_index("s")

    def motif_body(im_local, _):
        im = c * 6 + im_local
        last = im_local == 6
        im_out = jnp.where(last, 12 + c, im)

        _fill_zero_2d(rb0, B)
        for t in range(5):
            pltpu.sync_copy(rb0, acc.at[pl.ds(640 * s + B * t, B)])
        plsc.subcore_barrier()

        nsb_half = NSB_M1 // 2
        base_sb = jnp.where(last, 12 * NSB_M1 + c * nsb_half, im * NSB_M1)
        limit = jnp.where(last, nsb_half, NSB_M1)

        def body(i, _):
            sbl = s + 16 * i

            @pl.when(sbl < limit)
            def _():
                _process_superblock(h_hbm, col2, row2, val2,
                                    base_sb + sbl,
                                    col8, row8, val8, rb0, rb1,
                                    acc, sem_i, sem_g0, sem_g1,
                                    sem_s0, sem_s1)
            return 0
        lax.fori_loop(0, (NSB_M1 + 15) // 16, body, 0)

        plsc.subcore_barrier()
        pltpu.sync_copy(acc.at[pl.ds(640 * s, 640)],
                        mot_out.at[im_out, pl.ds(640 * s, 640)])
        plsc.subcore_barrier()
        return 0
    lax.fori_loop(0, 7, motif_body, 0)


def _motif_spmm(h, col2, row2, val2):
    mesh = plsc.VectorSubcoreMesh(core_axis_name="c", subcore_axis_name="s")
    f = functools.partial(
        pl.kernel,
        out_type=jax.ShapeDtypeStruct((NM + 1, NPAD, CH), _f32),
        mesh=mesh,
        scratch_types=[
            pltpu.VMEM((8, B), _i32),
            pltpu.VMEM((8, B), _i32),
            pltpu.VMEM((8, B), _f32),
            pltpu.VMEM((B, CH), _f32),
            pltpu.VMEM((B, CH), _f32),
            pltpu.VMEM_SHARED((NPAD, CH), _f32),
            pltpu.SemaphoreType.DMA,
            pltpu.SemaphoreType.DMA,
            pltpu.SemaphoreType.DMA,
            pltpu.SemaphoreType.DMA,
            pltpu.SemaphoreType.DMA,
        ],
    )(_motif_kernel)
    return f(h, col2, row2, val2)


def _attn_body(h_ref, mot_ref, wbig_ref, wa_ref, ba_ref, mb_ref, o_ref):
    parts = ([h_ref[...]] + [mot_ref[j] for j in range(NM - 1)]
             + [mot_ref[12] + mot_ref[13]])
    a = jnp.concatenate(parts, axis=1)
    cc = jnp.dot(a.astype(jnp.bfloat16), wbig_ref[...],
                 preferred_element_type=_f32)
    for i in range(1, NM + 1):
        mi = a[:, CH * i:CH * (i + 1)]
        mw = jnp.dot(mi, wa_ref[...], preferred_element_type=_f32) + ba_ref[...]
        ci = cc[:, D * (i - 1):D * i] + mb_ref[i - 1]
        att = 1.0 / (1.0 + jnp.exp(-jnp.sum(mw * ci, axis=1, keepdims=True)))
        o_ref[:, D * (i - 1):D * i] = att * (mw - ci)


def _attention(h, mot, wbig, wa, ba, mb):
    bn = 1000
    return pl.pallas_call(
        _attn_body,
        grid=(N // bn,),
        in_specs=[
            pl.BlockSpec((bn, CH), lambda i: (i, 0)),
            pl.BlockSpec((NM + 1, bn, CH), lambda i: (0, i, 0)),
            pl.BlockSpec(((NM + 1) * CH, NM * D), lambda i: (0, 0)),
            pl.BlockSpec((CH, D), lambda i: (0, 0)),
            pl.BlockSpec((D,), lambda i: (0,)),
            pl.BlockSpec((NM, D), lambda i: (0, 0)),
        ],
        out_specs=pl.BlockSpec((bn, NM * D), lambda i: (i, 0)),
        out_shape=jax.ShapeDtypeStruct((N, NM * D), _f32),
    )(h, mot, wbig, wa, ba, mb)


def kernel(x, edge_src, edge_dst, edge_w, motif_row, motif_col, motif_val,
           weight, root, bias, wa, ba, motif_weights, motif_biases):
    padE = EPAD - E
    tailB = ROWS_B * B - R * EPAD
    r_off = (jnp.arange(R, dtype=_i32) * NPAD)[:, None]
    spread = (jnp.arange(padE, dtype=_i32) * 157) % N
    colb = jnp.concatenate(
        [jnp.concatenate([edge_src + r_off,
                          jnp.broadcast_to(spread, (R, padE))],
                         axis=1).reshape(-1),
         (jnp.arange(tailB, dtype=_i32) * 157) % N]).reshape(ROWS_B, B)
    rowb = jnp.concatenate(
        [jnp.concatenate([edge_dst, jnp.full((R, padE), DUMP, _i32)],
                         axis=1).reshape(-1),
         jnp.full((tailB,), DUMP, _i32)]).reshape(ROWS_B, B)
    valb = jnp.concatenate(
        [jnp.concatenate([edge_w, jnp.zeros((R, padE), _f32)],
                         axis=1).reshape(-1),
         jnp.zeros((tailB,), _f32)]).reshape(ROWS_B, B)

    padM = EPAD_M - M
    spreadM = (jnp.arange(padM, dtype=_i32) * 157) % N
    colm = jnp.concatenate(
        [motif_col, jnp.broadcast_to(spreadM, (NM, padM))],
        axis=1).reshape(NM * ROWS_M1, B)
    rowm = jnp.concatenate(
        [motif_row, jnp.full((NM, padM), DUMP, _i32)],
        axis=1).reshape(NM * ROWS_M1, B)
    valm = jnp.concatenate(
        [motif_val, jnp.zeros((NM, padM), _f32)],
        axis=1).reshape(NM * ROWS_M1, B)

    wbig_cols = []
    zero_blk = jnp.zeros((CH, D), _f32)
    for i in range(1, NM + 1):
        wi = motif_weights[i - 1]
        wbig_cols.append(jnp.concatenate(
            [wi[:CH * i], zero_blk, wi[CH * i:]], axis=0))
    wbig = jnp.concatenate(wbig_cols, axis=1).astype(jnp.bfloat16)

    w4 = jnp.concatenate([weight, root[None]], axis=0)
    x_pad = jnp.concatenate([x, jnp.zeros((NPAD - N, CH), _f32)], axis=0)

    xt4 = _compute_xt(x_pad, w4)
    xt_flat = xt4.reshape(4 * NPAD, CH)
    hp, degp = _relation_aggregate(xt_flat, colb, rowb, valb)
    h = _combine(hp, degp, xt4, bias)
    mot = _motif_spmm(h, colm, rowm, valm)
    return _attention(h, mot, wbig, wa, ba, motif_biases)


# speedup vs baseline: 1.0091x; 1.0091x over previous
import functools

import jax
import jax.numpy as jnp
from jax import lax
from jax.experimental import pallas as pl
from jax.experimental.pallas import tpu as pltpu
from jax.experimental.pallas import tpu_sc as plsc

N = 10000
NPAD = 10240
R = 3
E = 200000
M = 200000
B = 128
CH = 128
D = 64
NM = 13
DUMP = 10016

EPAD = 200064
ROWS_B = 4696
NSB_B = ROWS_B // 8

EPAD_M = 200704
ROWS_M1 = EPAD_M // B
NSB_M1 = ROWS_M1 // 8

_f32 = jnp.float32
_i32 = jnp.int32


def _xt_body(x_ref, w_ref, o_ref):
    o_ref[0] = jnp.dot(x_ref[...], w_ref[0], preferred_element_type=_f32)


def _compute_xt(x_pad, w4):
    return pl.pallas_call(
        _xt_body,
        grid=(4, 10),
        in_specs=[
            pl.BlockSpec((NPAD // 10, CH), lambda i, j: (j, 0)),
            pl.BlockSpec((1, CH, CH), lambda i, j: (i, 0, 0)),
        ],
        out_specs=pl.BlockSpec((1, NPAD // 10, CH), lambda i, j: (i, j, 0)),
        out_shape=jax.ShapeDtypeStruct((4, NPAD, CH), _f32),
    )(x_pad, w4)


def _fill_zero_2d(buf, rows):
    def body(e, _):
        for j in range(CH // 16):
            buf[e, pl.ds(16 * j, 16)] = jnp.zeros((16,), _f32)
        return 0
    lax.fori_loop(0, rows, body, 0)


def _fill_zero_1d(buf, n):
    def body(g, _):
        buf[pl.ds(16 * g, 16)] = jnp.zeros((16,), _f32)
        return 0
    lax.fori_loop(0, n // 16, body, 0)


def _scale_chunk(rows_ref, val8, j):
    def body(e, _):
        e_hi = (e // 16) * 16
        vals16 = val8[j, pl.ds(e_hi, 16)]
        w16 = vals16.at[jnp.full((16,), 0, _i32) + (e - e_hi)].get(
            mode="promise_in_bounds")
        for c in range(CH // 16):
            sl = pl.ds(16 * c, 16)
            rows_ref[e, sl] = rows_ref[e, sl] * w16
        return 0
    lax.fori_loop(0, B, body, 0, unroll=4)


def _process_superblock(table, col2, row2, val2, sbk,
                        col8, row8, val8, rb0, rb1,
                        acc, sem_i, sem_g0, sem_g1, sem_s0, sem_s1,
                        degs=None, onesv=None):
    i1 = pltpu.async_copy(col2.at[pl.ds(8 * sbk, 8)], col8, sem_i)
    i2 = pltpu.async_copy(row2.at[pl.ds(8 * sbk, 8)], row8, sem_i)
    i3 = pltpu.async_copy(val2.at[pl.ds(8 * sbk, 8)], val8, sem_i)
    i1.wait()
    i2.wait()
    i3.wait()
    rbs = (rb0, rb1)
    sgs = (sem_g0, sem_g1)
    sss = (sem_s0, sem_s1)
    gat = [None] * 8
    sca = [None] * 8
    gat[0] = pltpu.async_copy(table.at[col8.at[0]], rbs[0], sgs[0])
    for j in range(8):
        b = j % 2
        if j >= 1:
            sca[j - 1].wait()
        if j < 7:
            gat[j + 1] = pltpu.async_copy(
                table.at[col8.at[j + 1]], rbs[1 - b], sgs[1 - b])
        gat[j].wait()
        _scale_chunk(rbs[b], val8, j)
        sca[j] = pltpu.async_copy(rbs[b], acc.at[row8.at[j]], sss[b],
                                  add=True)
        if degs is not None:
            pltpu.sync_copy(onesv, degs.at[row8.at[j]], add=True)
    sca[7].wait()


def _relation_kernel(xt_hbm, col2, row2, val2, hp_out, deg_out,
                     col8, row8, val8, rb0, rb1, onesv,
                     acc, degs, sem_i, sem_g0, sem_g1, sem_s0, sem_s1):
    c = lax.axis_index("c")
    s = lax.axis_index("s")
    w = s * 2 + c

    _fill_zero_2d(rb0, B)
    for t in range(5):
        pltpu.sync_copy(rb0, acc.at[pl.ds(640 * s + B * t, B)])
    _fill_zero_1d(onesv, B)
    for t in range(5):
        pltpu.sync_copy(onesv, degs.at[pl.ds(640 * s + B * t, B)])
    for g in range(B // 16):
        onesv[pl.ds(16 * g, 16)] = jnp.ones((16,), _f32)
    plsc.subcore_barrier()

    def body(i, _):
        sbk = w + 32 * i

        @pl.when(sbk < NSB_B)
        def _():
            _process_superblock(xt_hbm, col2, row2, val2, sbk,
                                col8, row8, val8, rb0, rb1,
                                acc, sem_i, sem_g0, sem_g1, sem_s0, sem_s1,
                                degs=degs, onesv=onesv)
        return 0
    lax.fori_loop(0, (NSB_B + 31) // 32, body, 0)

    plsc.subcore_barrier()
    pltpu.sync_copy(acc.at[pl.ds(640 * s, 640)],
                    hp_out.at[c, pl.ds(640 * s, 640)])
    pltpu.sync_copy(degs.at[pl.ds(640 * s, 640)],
                    deg_out.at[pl.ds(c * NPAD + 640 * s, 640)])


def _relation_aggregate(xt_flat, col2, row2, val2):
    mesh = plsc.VectorSubcoreMesh(core_axis_name="c", subcore_axis_name="s")
    f = functools.partial(
        pl.kernel,
        out_type=[
            jax.ShapeDtypeStruct((2, NPAD, CH), _f32),
            jax.ShapeDtypeStruct((2 * NPAD,), _f32),
        ],
        mesh=mesh,
        scratch_types=[
            pltpu.VMEM((8, B), _i32),
            pltpu.VMEM((8, B), _i32),
            pltpu.VMEM((8, B), _f32),
            pltpu.VMEM((B, CH), _f32),
            pltpu.VMEM((B, CH), _f32),
            pltpu.VMEM((B,), _f32),
            pltpu.VMEM_SHARED((NPAD, CH), _f32),
            pltpu.VMEM_SHARED((NPAD,), _f32),
            pltpu.SemaphoreType.DMA,
            pltpu.SemaphoreType.DMA,
            pltpu.SemaphoreType.DMA,
            pltpu.SemaphoreType.DMA,
            pltpu.SemaphoreType.DMA,
        ],
    )(_relation_kernel)
    return f(xt_flat, col2, row2, val2)


def _combine_body(hp_ref, deg_ref, xt4_ref, b_ref, o_ref):
    i = pl.program_id(0)
    bn = NPAD // 10
    deg = (deg_ref[pl.ds(i * bn, bn)]
           + deg_ref[pl.ds(NPAD + i * bn, bn)])
    norm = jnp.where(deg > 0, 1.0 / jnp.maximum(deg, 1.0), 0.0)
    h = (hp_ref[0] + hp_ref[1]) * norm[:, None] + xt4_ref[0] + b_ref[...]
    o_ref[...] = h


def _combine(hp, degp, xt4, bias):
    bn = NPAD // 10
    return pl.pallas_call(
        _combine_body,
        grid=(10,),
        in_specs=[
            pl.BlockSpec((2, bn, CH), lambda i: (0, i, 0)),
            pl.BlockSpec((2 * NPAD,), lambda i: (0,)),
            pl.BlockSpec((1, bn, CH), lambda i: (3, i, 0)),
            pl.BlockSpec((CH,), lambda i: (0,)),
        ],
        out_specs=pl.BlockSpec((bn, CH), lambda i: (i, 0)),
        out_shape=jax.ShapeDtypeStruct((NPAD, CH), _f32),
    )(hp, degp, xt4, bias)


def _motif_kernel(h_hbm, col2, row2, val2, mot_out,
                  col8, row8, val8, rb0, rb1,
                  acc, sem_i, sem_g0, sem_g1, sem_s0, sem_s1):
    c = lax.axis_index("c")
    s = lax.axis_index("s")

    def motif_body(im_local, _):
        im = c * 6 + im_local
        last = im_local == 6
        im_out = jnp.where(last, 12 + c, im)

        _fill_zero_2d(rb0, B)
        for t in range(5):
            pltpu.sync_copy(rb0, acc.at[pl.ds(640 * s + B * t, B)])
        plsc.subcore_barrier()

        nsb_half = NSB_M1 // 2
        base_sb = jnp.where(last, 12 * NSB_M1 + c * nsb_half, im * NSB_M1)
        limit = jnp.where(last, nsb_half, NSB_M1)

        def body(i, _):
            sbl = s + 16 * i

            @pl.when(sbl < limit)
            def _():
                _process_superblock(h_hbm, col2, row2, val2,
                                    base_sb + sbl,
                                    col8, row8, val8, rb0, rb1,
                                    acc, sem_i, sem_g0, sem_g1,
                                    sem_s0, sem_s1)
            return 0
        lax.fori_loop(0, (NSB_M1 + 15) // 16, body, 0)

        plsc.subcore_barrier()
        pltpu.sync_copy(acc.at[pl.ds(640 * s, 640)],
                        mot_out.at[im_out, pl.ds(640 * s, 640)])
        plsc.subcore_barrier()
        return 0
    lax.fori_loop(0, 7, motif_body, 0)


def _motif_spmm(h, col2, row2, val2):
    mesh = plsc.VectorSubcoreMesh(core_axis_name="c", subcore_axis_name="s")
    f = functools.partial(
        pl.kernel,
        out_type=jax.ShapeDtypeStruct((NM + 1, NPAD, CH), _f32),
        mesh=mesh,
        scratch_types=[
            pltpu.VMEM((8, B), _i32),
            pltpu.VMEM((8, B), _i32),
            pltpu.VMEM((8, B), _f32),
            pltpu.VMEM((B, CH), _f32),
            pltpu.VMEM((B, CH), _f32),
            pltpu.VMEM_SHARED((NPAD, CH), _f32),
            pltpu.SemaphoreType.DMA,
            pltpu.SemaphoreType.DMA,
            pltpu.SemaphoreType.DMA,
            pltpu.SemaphoreType.DMA,
            pltpu.SemaphoreType.DMA,
        ],
    )(_motif_kernel)
    return f(h, col2, row2, val2)


def _attn_body(h_ref, mot_ref, wbig_ref, wa_ref, ba_ref, mb_ref, o_ref):
    parts = ([h_ref[...]] + [mot_ref[j] for j in range(NM - 1)]
             + [mot_ref[12] + mot_ref[13]])
    a = jnp.concatenate(parts, axis=1)
    cc = jnp.dot(a.astype(jnp.bfloat16), wbig_ref[...],
                 preferred_element_type=_f32)
    for i in range(1, NM + 1):
        mi = a[:, CH * i:CH * (i + 1)]
        mw = jnp.dot(mi, wa_ref[...], preferred_element_type=_f32) + ba_ref[...]
        ci = cc[:, D * (i - 1):D * i] + mb_ref[i - 1]
        att = 1.0 / (1.0 + jnp.exp(-jnp.sum(mw * ci, axis=1, keepdims=True)))
        o_ref[:, D * (i - 1):D * i] = att * (mw - ci)


def _attention(h, mot, wbig, wa, ba, mb):
    bn = 1000
    return pl.pallas_call(
        _attn_body,
        grid=(N // bn,),
        in_specs=[
            pl.BlockSpec((bn, CH), lambda i: (i, 0)),
            pl.BlockSpec((NM + 1, bn, CH), lambda i: (0, i, 0)),
            pl.BlockSpec(((NM + 1) * CH, NM * D), lambda i: (0, 0)),
            pl.BlockSpec((CH, D), lambda i: (0, 0)),
            pl.BlockSpec((D,), lambda i: (0,)),
            pl.BlockSpec((NM, D), lambda i: (0, 0)),
        ],
        out_specs=pl.BlockSpec((bn, NM * D), lambda i: (i, 0)),
        out_shape=jax.ShapeDtypeStruct((N, NM * D), _f32),
    )(h, mot, wbig, wa, ba, mb)


def kernel(x, edge_src, edge_dst, edge_w, motif_row, motif_col, motif_val,
           weight, root, bias, wa, ba, motif_weights, motif_biases):
    padE = EPAD - E
    tailB = ROWS_B * B - R * EPAD
    r_off = (jnp.arange(R, dtype=_i32) * NPAD)[:, None]
    spread = (jnp.arange(padE, dtype=_i32) * 157) % N
    colb = jnp.concatenate(
        [jnp.concatenate([edge_src + r_off,
                          jnp.broadcast_to(spread, (R, padE))],
                         axis=1).reshape(-1),
         (jnp.arange(tailB, dtype=_i32) * 157) % N]).reshape(ROWS_B, B)
    rowb = jnp.concatenate(
        [jnp.concatenate([edge_dst, jnp.full((R, padE), DUMP, _i32)],
                         axis=1).reshape(-1),
         jnp.full((tailB,), DUMP, _i32)]).reshape(ROWS_B, B)
    valb = jnp.concatenate(
        [jnp.concatenate([edge_w, jnp.zeros((R, padE), _f32)],
                         axis=1).reshape(-1),
         jnp.zeros((tailB,), _f32)]).reshape(ROWS_B, B)

    padM = EPAD_M - M
    spreadM = (jnp.arange(padM, dtype=_i32) * 157) % N
    colm = jnp.concatenate(
        [motif_col, jnp.broadcast_to(spreadM, (NM, padM))],
        axis=1).reshape(NM * ROWS_M1, B)
    rowm = jnp.concatenate(
        [motif_row, jnp.full((NM, padM), DUMP, _i32)],
        axis=1).reshape(NM * ROWS_M1, B)
    valm = jnp.concatenate(
        [motif_val, jnp.zeros((NM, padM), _f32)],
        axis=1).reshape(NM * ROWS_M1, B)

    wbig_cols = []
    zero_blk = jnp.zeros((CH, D), _f32)
    for i in range(1, NM + 1):
        wi = motif_weights[i - 1]
        wbig_cols.append(jnp.concatenate(
            [wi[:CH * i], zero_blk, wi[CH * i:]], axis=0))
    wbig = jnp.concatenate(wbig_cols, axis=1).astype(jnp.bfloat16)

    w4 = jnp.concatenate([weight, root[None]], axis=0)
    x_pad = jnp.concatenate([x, jnp.zeros((NPAD - N, CH), _f32)], axis=0)

    xt4 = _compute_xt(x_pad, w4)
    xt_flat = xt4.reshape(4 * NPAD, CH)
    hp, degp = _relation_aggregate(xt_flat, colb, rowb, valb)
    h = _combine(hp, degp, xt4, bias)
    mot = _motif_spmm(h, colm, rowm, valm)
    return _attention(h, mot, wbig, wa, ba, motif_biases)


# speedup vs baseline: 1.2498x; 1.2385x over previous
import functools

import jax
import jax.numpy as jnp
from jax import lax
from jax.experimental import pallas as pl
from jax.experimental.pallas import tpu as pltpu
from jax.experimental.pallas import tpu_sc as plsc

N = 10000
NPAD = 10240
R = 3
E = 200000
M = 200000
B = 128
CH = 128
D = 64
NM = 13
DUMP = 10016

EPAD = 200064
ROWS_B = 4696
NSB_B = ROWS_B // 8

EPAD_M = 200704
ROWS_M1 = EPAD_M // B
NSB_M1 = ROWS_M1 // 8

_f32 = jnp.float32
_i32 = jnp.int32


def _xt_body(x_ref, w_ref, o_ref):
    o_ref[0] = jnp.dot(x_ref[...], w_ref[0], preferred_element_type=_f32)


def _compute_xt(x_pad, w4):
    return pl.pallas_call(
        _xt_body,
        grid=(4, 10),
        in_specs=[
            pl.BlockSpec((NPAD // 10, CH), lambda i, j: (j, 0)),
            pl.BlockSpec((1, CH, CH), lambda i, j: (i, 0, 0)),
        ],
        out_specs=pl.BlockSpec((1, NPAD // 10, CH), lambda i, j: (i, j, 0)),
        out_shape=jax.ShapeDtypeStruct((4, NPAD, CH), _f32),
    )(x_pad, w4)


def _fill_zero_2d(buf, rows):
    def body(e, _):
        for j in range(CH // 16):
            buf[e, pl.ds(16 * j, 16)] = jnp.zeros((16,), _f32)
        return 0
    lax.fori_loop(0, rows, body, 0)


def _fill_zero_1d(buf, n):
    def body(g, _):
        buf[pl.ds(16 * g, 16)] = jnp.zeros((16,), _f32)
        return 0
    lax.fori_loop(0, n // 16, body, 0)


def _scale_chunk(rows_ref, val8, j):
    lane_idx = [jnp.full((16,), lane, _i32) for lane in range(16)]

    def body(g, _):
        vals16 = val8[j, pl.ds(16 * g, 16)]
        base = 16 * g
        w16s = [vals16.at[lane_idx[lane]].get(mode="promise_in_bounds")
                for lane in range(16)]
        for lane in range(16):
            e = base + lane
            for c in range(CH // 16):
                sl = pl.ds(16 * c, 16)
                rows_ref[e, sl] = rows_ref[e, sl] * w16s[lane]
        return 0
    lax.fori_loop(0, B // 16, body, 0)


def _process_superblock(table, col2, row2, val2, sbk,
                        col8, row8, val8, rb0, rb1,
                        acc, sem_i, sem_g0, sem_g1, sem_s0, sem_s1,
                        degs=None, onesv=None):
    i1 = pltpu.async_copy(col2.at[pl.ds(8 * sbk, 8)], col8, sem_i)
    i2 = pltpu.async_copy(row2.at[pl.ds(8 * sbk, 8)], row8, sem_i)
    i3 = pltpu.async_copy(val2.at[pl.ds(8 * sbk, 8)], val8, sem_i)
    i1.wait()
    i2.wait()
    i3.wait()
    rbs = (rb0, rb1)
    sgs = (sem_g0, sem_g1)
    sss = (sem_s0, sem_s1)
    gat = [None] * 8
    sca = [None] * 8
    gat[0] = pltpu.async_copy(table.at[col8.at[0]], rbs[0], sgs[0])
    for j in range(8):
        b = j % 2
        if j >= 1:
            sca[j - 1].wait()
        if j < 7:
            gat[j + 1] = pltpu.async_copy(
                table.at[col8.at[j + 1]], rbs[1 - b], sgs[1 - b])
        gat[j].wait()
        _scale_chunk(rbs[b], val8, j)
        sca[j] = pltpu.async_copy(rbs[b], acc.at[row8.at[j]], sss[b],
                                  add=True)
        if degs is not None:
            pltpu.sync_copy(onesv, degs.at[row8.at[j]], add=True)
    sca[7].wait()


def _relation_kernel(xt_hbm, col2, row2, val2, hp_out, deg_out,
                     col8, row8, val8, rb0, rb1, onesv,
                     acc, degs, sem_i, sem_g0, sem_g1, sem_s0, sem_s1):
    c = lax.axis_index("c")
    s = lax.axis_index("s")
    w = s * 2 + c

    _fill_zero_2d(rb0, B)
    for t in range(5):
        pltpu.sync_copy(rb0, acc.at[pl.ds(640 * s + B * t, B)])
    _fill_zero_1d(onesv, B)
    for t in range(5):
        pltpu.sync_copy(onesv, degs.at[pl.ds(640 * s + B * t, B)])
    for g in range(B // 16):
        onesv[pl.ds(16 * g, 16)] = jnp.ones((16,), _f32)
    plsc.subcore_barrier()

    def body(i, _):
        sbk = w + 32 * i

        @pl.when(sbk < NSB_B)
        def _():
            _process_superblock(xt_hbm, col2, row2, val2, sbk,
                                col8, row8, val8, rb0, rb1,
                                acc, sem_i, sem_g0, sem_g1, sem_s0, sem_s1,
                                degs=degs, onesv=onesv)
        return 0
    lax.fori_loop(0, (NSB_B + 31) // 32, body, 0)

    plsc.subcore_barrier()
    pltpu.sync_copy(acc.at[pl.ds(640 * s, 640)],
                    hp_out.at[c, pl.ds(640 * s, 640)])
    pltpu.sync_copy(degs.at[pl.ds(640 * s, 640)],
                    deg_out.at[pl.ds(c * NPAD + 640 * s, 640)])


def _relation_aggregate(xt_flat, col2, row2, val2):
    mesh = plsc.VectorSubcoreMesh(core_axis_name="c", subcore_axis_name="s")
    f = functools.partial(
        pl.kernel,
        out_type=[
            jax.ShapeDtypeStruct((2, NPAD, CH), _f32),
            jax.ShapeDtypeStruct((2 * NPAD,), _f32),
        ],
        mesh=mesh,
        scratch_types=[
            pltpu.VMEM((8, B), _i32),
            pltpu.VMEM((8, B), _i32),
            pltpu.VMEM((8, B), _f32),
            pltpu.VMEM((B, CH), _f32),
            pltpu.VMEM((B, CH), _f32),
            pltpu.VMEM((B,), _f32),
            pltpu.VMEM_SHARED((NPAD, CH), _f32),
            pltpu.VMEM_SHARED((NPAD,), _f32),
            pltpu.SemaphoreType.DMA,
            pltpu.SemaphoreType.DMA,
            pltpu.SemaphoreType.DMA,
            pltpu.SemaphoreType.DMA,
            pltpu.SemaphoreType.DMA,
        ],
    )(_relation_kernel)
    return f(xt_flat, col2, row2, val2)


def _combine_body(hp_ref, deg_ref, xt4_ref, b_ref, o_ref):
    i = pl.program_id(0)
    bn = NPAD // 10
    deg = (deg_ref[pl.ds(i * bn, bn)]
           + deg_ref[pl.ds(NPAD + i * bn, bn)])
    norm = jnp.where(deg > 0, 1.0 / jnp.maximum(deg, 1.0), 0.0)
    h = (hp_ref[0] + hp_ref[1]) * norm[:, None] + xt4_ref[0] + b_ref[...]
    o_ref[...] = h


def _combine(hp, degp, xt4, bias):
    bn = NPAD // 10
    return pl.pallas_call(
        _combine_body,
        grid=(10,),
        in_specs=[
            pl.BlockSpec((2, bn, CH), lambda i: (0, i, 0)),
            pl.BlockSpec((2 * NPAD,), lambda i: (0,)),
            pl.BlockSpec((1, bn, CH), lambda i: (3, i, 0)),
            pl.BlockSpec((CH,), lambda i: (0,)),
        ],
        out_specs=pl.BlockSpec((bn, CH), lambda i: (i, 0)),
        out_shape=jax.ShapeDtypeStruct((NPAD, CH), _f32),
    )(hp, degp, xt4, bias)


def _motif_kernel(h_hbm, col2, row2, val2, mot_out,
                  col8, row8, val8, rb0, rb1,
                  acc, sem_i, sem_g0, sem_g1, sem_s0, sem_s1):
    c = lax.axis_index("c")
    s = lax.axis_index("s")

    def motif_body(im_local, _):
        im = c * 6 + im_local
        last = im_local == 6
        im_out = jnp.where(last, 12 + c, im)

        _fill_zero_2d(rb0, B)
        for t in range(5):
            pltpu.sync_copy(rb0, acc.at[pl.ds(640 * s + B * t, B)])
        plsc.subcore_barrier()

        nsb_half = NSB_M1 // 2
        base_sb = jnp.where(last, 12 * NSB_M1 + c * nsb_half, im * NSB_M1)
        limit = jnp.where(last, nsb_half, NSB_M1)

        def body(i, _):
            sbl = s + 16 * i

            @pl.when(sbl < limit)
            def _():
                _process_superblock(h_hbm, col2, row2, val2,
                                    base_sb + sbl,
                                    col8, row8, val8, rb0, rb1,
                                    acc, sem_i, sem_g0, sem_g1,
                                    sem_s0, sem_s1)
            return 0
        lax.fori_loop(0, (NSB_M1 + 15) // 16, body, 0)

        plsc.subcore_barrier()
        pltpu.sync_copy(acc.at[pl.ds(640 * s, 640)],
                        mot_out.at[im_out, pl.ds(640 * s, 640)])
        plsc.subcore_barrier()
        return 0
    lax.fori_loop(0, 7, motif_body, 0)


def _motif_spmm(h, col2, row2, val2):
    mesh = plsc.VectorSubcoreMesh(core_axis_name="c", subcore_axis_name="s")
    f = functools.partial(
        pl.kernel,
        out_type=jax.ShapeDtypeStruct((NM + 1, NPAD, CH), _f32),
        mesh=mesh,
        scratch_types=[
            pltpu.VMEM((8, B), _i32),
            pltpu.VMEM((8, B), _i32),
            pltpu.VMEM((8, B), _f32),
            pltpu.VMEM((B, CH), _f32),
            pltpu.VMEM((B, CH), _f32),
            pltpu.VMEM_SHARED((NPAD, CH), _f32),
            pltpu.SemaphoreType.DMA,
            pltpu.SemaphoreType.DMA,
            pltpu.SemaphoreType.DMA,
            pltpu.SemaphoreType.DMA,
            pltpu.SemaphoreType.DMA,
        ],
    )(_motif_kernel)
    return f(h, col2, row2, val2)


def _attn_body(h_ref, mot_ref, wbig_ref, wa_ref, ba_ref, mb_ref, o_ref):
    parts = ([h_ref[...]] + [mot_ref[j] for j in range(NM - 1)]
             + [mot_ref[12] + mot_ref[13]])
    a = jnp.concatenate(parts, axis=1)
    cc = jnp.dot(a.astype(jnp.bfloat16), wbig_ref[...],
                 preferred_element_type=_f32)
    for i in range(1, NM + 1):
        mi = a[:, CH * i:CH * (i + 1)]
        mw = jnp.dot(mi, wa_ref[...], preferred_element_type=_f32) + ba_ref[...]
        ci = cc[:, D * (i - 1):D * i] + mb_ref[i - 1]
        att = 1.0 / (1.0 + jnp.exp(-jnp.sum(mw * ci, axis=1, keepdims=True)))
        o_ref[:, D * (i - 1):D * i] = att * (mw - ci)


def _attention(h, mot, wbig, wa, ba, mb):
    bn = 1000
    return pl.pallas_call(
        _attn_body,
        grid=(N // bn,),
        in_specs=[
            pl.BlockSpec((bn, CH), lambda i: (i, 0)),
            pl.BlockSpec((NM + 1, bn, CH), lambda i: (0, i, 0)),
            pl.BlockSpec(((NM + 1) * CH, NM * D), lambda i: (0, 0)),
            pl.BlockSpec((CH, D), lambda i: (0, 0)),
            pl.BlockSpec((D,), lambda i: (0,)),
            pl.BlockSpec((NM, D), lambda i: (0, 0)),
        ],
        out_specs=pl.BlockSpec((bn, NM * D), lambda i: (i, 0)),
        out_shape=jax.ShapeDtypeStruct((N, NM * D), _f32),
    )(h, mot, wbig, wa, ba, mb)


def kernel(x, edge_src, edge_dst, edge_w, motif_row, motif_col, motif_val,
           weight, root, bias, wa, ba, motif_weights, motif_biases):
    padE = EPAD - E
    tailB = ROWS_B * B - R * EPAD
    r_off = (jnp.arange(R, dtype=_i32) * NPAD)[:, None]
    spread = (jnp.arange(padE, dtype=_i32) * 157) % N
    colb = jnp.concatenate(
        [jnp.concatenate([edge_src + r_off,
                          jnp.broadcast_to(spread, (R, padE))],
                         axis=1).reshape(-1),
         (jnp.arange(tailB, dtype=_i32) * 157) % N]).reshape(ROWS_B, B)
    rowb = jnp.concatenate(
        [jnp.concatenate([edge_dst, jnp.full((R, padE), DUMP, _i32)],
                         axis=1).reshape(-1),
         jnp.full((tailB,), DUMP, _i32)]).reshape(ROWS_B, B)
    valb = jnp.concatenate(
        [jnp.concatenate([edge_w, jnp.zeros((R, padE), _f32)],
                         axis=1).reshape(-1),
         jnp.zeros((tailB,), _f32)]).reshape(ROWS_B, B)

    padM = EPAD_M - M
    spreadM = (jnp.arange(padM, dtype=_i32) * 157) % N
    colm = jnp.concatenate(
        [motif_col, jnp.broadcast_to(spreadM, (NM, padM))],
        axis=1).reshape(NM * ROWS_M1, B)
    rowm = jnp.concatenate(
        [motif_row, jnp.full((NM, padM), DUMP, _i32)],
        axis=1).reshape(NM * ROWS_M1, B)
    valm = jnp.concatenate(
        [motif_val, jnp.zeros((NM, padM), _f32)],
        axis=1).reshape(NM * ROWS_M1, B)

    wbig_cols = []
    zero_blk = jnp.zeros((CH, D), _f32)
    for i in range(1, NM + 1):
        wi = motif_weights[i - 1]
        wbig_cols.append(jnp.concatenate(
            [wi[:CH * i], zero_blk, wi[CH * i:]], axis=0))
    wbig = jnp.concatenate(wbig_cols, axis=1).astype(jnp.bfloat16)

    w4 = jnp.concatenate([weight, root[None]], axis=0)
    x_pad = jnp.concatenate([x, jnp.zeros((NPAD - N, CH), _f32)], axis=0)

    xt4 = _compute_xt(x_pad, w4)
    xt_flat = xt4.reshape(4 * NPAD, CH)
    hp, degp = _relation_aggregate(xt_flat, colb, rowb, valb)
    h = _combine(hp, degp, xt4, bias)
    mot = _motif_spmm(h, colm, rowm, valm)
    return _attention(h, mot, wbig, wa, ba, motif_biases)


# speedup vs baseline: 1.3133x; 1.0509x over previous
import functools

import jax
import jax.numpy as jnp
from jax import lax
from jax.experimental import pallas as pl
from jax.experimental.pallas import tpu as pltpu
from jax.experimental.pallas import tpu_sc as plsc

N = 10000
NPAD = 10240
R = 3
E = 200000
M = 200000
B = 128
CH = 128
D = 64
NM = 13
DUMP = 10016

EPAD = 200064
ROWS_B = 4696
NSB_B = ROWS_B // 8

EPAD_M = 200704
ROWS_M1 = EPAD_M // B
NSB_M1 = ROWS_M1 // 8

_f32 = jnp.float32
_i32 = jnp.int32


def _xt_body(x_ref, w_ref, o_ref):
    o_ref[0] = jnp.dot(x_ref[...], w_ref[0], preferred_element_type=_f32)


def _compute_xt(x_pad, w4):
    return pl.pallas_call(
        _xt_body,
        grid=(4, 10),
        in_specs=[
            pl.BlockSpec((NPAD // 10, CH), lambda i, j: (j, 0)),
            pl.BlockSpec((1, CH, CH), lambda i, j: (i, 0, 0)),
        ],
        out_specs=pl.BlockSpec((1, NPAD // 10, CH), lambda i, j: (i, j, 0)),
        out_shape=jax.ShapeDtypeStruct((4, NPAD, CH), _f32),
    )(x_pad, w4)


def _fill_zero_2d(buf, rows):
    def body(e, _):
        for j in range(CH // 16):
            buf[e, pl.ds(16 * j, 16)] = jnp.zeros((16,), _f32)
        return 0
    lax.fori_loop(0, rows, body, 0)


def _fill_zero_1d(buf, n):
    def body(g, _):
        buf[pl.ds(16 * g, 16)] = jnp.zeros((16,), _f32)
        return 0
    lax.fori_loop(0, n // 16, body, 0)


def _scale_chunk(rows_ref, val8, j):
    lane_idx = [jnp.full((16,), lane, _i32) for lane in range(16)]

    def body(g, _):
        vals16 = val8[j, pl.ds(16 * g, 16)]
        base = 16 * g
        w16s = [vals16.at[lane_idx[lane]].get(mode="promise_in_bounds")
                for lane in range(16)]
        for lane in range(16):
            e = base + lane
            for c in range(CH // 16):
                sl = pl.ds(16 * c, 16)
                rows_ref[e, sl] = rows_ref[e, sl] * w16s[lane]
        return 0
    lax.fori_loop(0, B // 16, body, 0)


def _process_superblock(table, col2, row2, val2, sbk,
                        col8, row8, val8, rb0, rb1,
                        acc, sem_i, sem_g0, sem_g1, sem_s0, sem_s1,
                        degs=None, onesv=None):
    i1 = pltpu.async_copy(col2.at[pl.ds(8 * sbk, 8)], col8, sem_i)
    i2 = pltpu.async_copy(row2.at[pl.ds(8 * sbk, 8)], row8, sem_i)
    i3 = pltpu.async_copy(val2.at[pl.ds(8 * sbk, 8)], val8, sem_i)
    i1.wait()
    i2.wait()
    i3.wait()
    rbs = (rb0, rb1)
    sgs = (sem_g0, sem_g1)
    sss = (sem_s0, sem_s1)
    gat = [None] * 8
    sca = [None] * 8
    scd = [None] * 8
    gat[0] = pltpu.async_copy(table.at[col8.at[0]], rbs[0], sgs[0])
    for j in range(8):
        b = j % 2
        if j >= 1:
            sca[j - 1].wait()
            if degs is not None:
                scd[j - 1].wait()
        if j < 7:
            gat[j + 1] = pltpu.async_copy(
                table.at[col8.at[j + 1]], rbs[1 - b], sgs[1 - b])
        gat[j].wait()
        _scale_chunk(rbs[b], val8, j)
        sca[j] = pltpu.async_copy(rbs[b], acc.at[row8.at[j]], sss[b],
                                  add=True)
        if degs is not None:
            scd[j] = pltpu.async_copy(onesv, degs.at[row8.at[j]], sss[b],
                                      add=True)
    sca[7].wait()
    if degs is not None:
        scd[7].wait()


def _relation_kernel(xt_hbm, col2, row2, val2, hp_out, deg_out,
                     col8, row8, val8, rb0, rb1, onesv,
                     acc, degs, sem_i, sem_g0, sem_g1, sem_s0, sem_s1):
    c = lax.axis_index("c")
    s = lax.axis_index("s")
    w = s * 2 + c

    _fill_zero_2d(rb0, B)
    for t in range(5):
        pltpu.sync_copy(rb0, acc.at[pl.ds(640 * s + B * t, B)])
    _fill_zero_1d(onesv, B)
    for t in range(5):
        pltpu.sync_copy(onesv, degs.at[pl.ds(640 * s + B * t, B)])
    for g in range(B // 16):
        onesv[pl.ds(16 * g, 16)] = jnp.ones((16,), _f32)
    plsc.subcore_barrier()

    def body(i, _):
        sbk = w + 32 * i

        @pl.when(sbk < NSB_B)
        def _():
            _process_superblock(xt_hbm, col2, row2, val2, sbk,
                                col8, row8, val8, rb0, rb1,
                                acc, sem_i, sem_g0, sem_g1, sem_s0, sem_s1,
                                degs=degs, onesv=onesv)
        return 0
    lax.fori_loop(0, (NSB_B + 31) // 32, body, 0)

    plsc.subcore_barrier()
    pltpu.sync_copy(acc.at[pl.ds(640 * s, 640)],
                    hp_out.at[c, pl.ds(640 * s, 640)])
    pltpu.sync_copy(degs.at[pl.ds(640 * s, 640)],
                    deg_out.at[pl.ds(c * NPAD + 640 * s, 640)])


def _relation_aggregate(xt_flat, col2, row2, val2):
    mesh = plsc.VectorSubcoreMesh(core_axis_name="c", subcore_axis_name="s")
    f = functools.partial(
        pl.kernel,
        out_type=[
            jax.ShapeDtypeStruct((2, NPAD, CH), _f32),
            jax.ShapeDtypeStruct((2 * NPAD,), _f32),
        ],
        mesh=mesh,
        scratch_types=[
            pltpu.VMEM((8, B), _i32),
            pltpu.VMEM((8, B), _i32),
            pltpu.VMEM((8, B), _f32),
            pltpu.VMEM((B, CH), _f32),
            pltpu.VMEM((B, CH), _f32),
            pltpu.VMEM((B,), _f32),
            pltpu.VMEM_SHARED((NPAD, CH), _f32),
            pltpu.VMEM_SHARED((NPAD,), _f32),
            pltpu.SemaphoreType.DMA,
            pltpu.SemaphoreType.DMA,
            pltpu.SemaphoreType.DMA,
            pltpu.SemaphoreType.DMA,
            pltpu.SemaphoreType.DMA,
        ],
    )(_relation_kernel)
    return f(xt_flat, col2, row2, val2)


def _combine_body(hp_ref, deg_ref, xt4_ref, b_ref, o_ref):
    i = pl.program_id(0)
    bn = NPAD // 10
    deg = (deg_ref[pl.ds(i * bn, bn)]
           + deg_ref[pl.ds(NPAD + i * bn, bn)])
    norm = jnp.where(deg > 0, 1.0 / jnp.maximum(deg, 1.0), 0.0)
    h = (hp_ref[0] + hp_ref[1]) * norm[:, None] + xt4_ref[0] + b_ref[...]
    o_ref[...] = h


def _combine(hp, degp, xt4, bias):
    bn = NPAD // 10
    return pl.pallas_call(
        _combine_body,
        grid=(10,),
        in_specs=[
            pl.BlockSpec((2, bn, CH), lambda i: (0, i, 0)),
            pl.BlockSpec((2 * NPAD,), lambda i: (0,)),
            pl.BlockSpec((1, bn, CH), lambda i: (3, i, 0)),
            pl.BlockSpec((CH,), lambda i: (0,)),
        ],
        out_specs=pl.BlockSpec((bn, CH), lambda i: (i, 0)),
        out_shape=jax.ShapeDtypeStruct((NPAD, CH), _f32),
    )(hp, degp, xt4, bias)


def _motif_kernel(h_hbm, col2, row2, val2, mot_out,
                  col8, row8, val8, rb0, rb1,
                  acc, sem_i, sem_g0, sem_g1, sem_s0, sem_s1):
    c = lax.axis_index("c")
    s = lax.axis_index("s")

    def motif_body(im_local, _):
        im = c * 6 + im_local
        last = im_local == 6
        im_out = jnp.where(last, 12 + c, im)

        _fill_zero_2d(rb0, B)
        for t in range(5):
            pltpu.sync_copy(rb0, acc.at[pl.ds(640 * s + B * t, B)])
        plsc.subcore_barrier()

        nsb_half = NSB_M1 // 2
        base_sb = jnp.where(last, 12 * NSB_M1 + c * nsb_half, im * NSB_M1)
        limit = jnp.where(last, nsb_half, NSB_M1)

        def body(i, _):
            sbl = s + 16 * i

            @pl.when(sbl < limit)
            def _():
                _process_superblock(h_hbm, col2, row2, val2,
                                    base_sb + sbl,
                                    col8, row8, val8, rb0, rb1,
                                    acc, sem_i, sem_g0, sem_g1,
                                    sem_s0, sem_s1)
            return 0
        lax.fori_loop(0, (NSB_M1 + 15) // 16, body, 0)

        plsc.subcore_barrier()
        pltpu.sync_copy(acc.at[pl.ds(640 * s, 640)],
                        mot_out.at[im_out, pl.ds(640 * s, 640)])
        plsc.subcore_barrier()
        return 0
    lax.fori_loop(0, 7, motif_body, 0)


def _motif_spmm(h, col2, row2, val2):
    mesh = plsc.VectorSubcoreMesh(core_axis_name="c", subcore_axis_name="s")
    f = functools.partial(
        pl.kernel,
        out_type=jax.ShapeDtypeStruct((NM + 1, NPAD, CH), _f32),
        mesh=mesh,
        scratch_types=[
            pltpu.VMEM((8, B), _i32),
            pltpu.VMEM((8, B), _i32),
            pltpu.VMEM((8, B), _f32),
            pltpu.VMEM((B, CH), _f32),
            pltpu.VMEM((B, CH), _f32),
            pltpu.VMEM_SHARED((NPAD, CH), _f32),
            pltpu.SemaphoreType.DMA,
            pltpu.SemaphoreType.DMA,
            pltpu.SemaphoreType.DMA,
            pltpu.SemaphoreType.DMA,
            pltpu.SemaphoreType.DMA,
        ],
    )(_motif_kernel)
    return f(h, col2, row2, val2)


def _attn_body(h_ref, mot_ref, wbig_ref, wa_ref, ba_ref, mb_ref, o_ref):
    parts = ([h_ref[...]] + [mot_ref[j] for j in range(NM - 1)]
             + [mot_ref[12] + mot_ref[13]])
    a = jnp.concatenate(parts, axis=1)
    cc = jnp.dot(a.astype(jnp.bfloat16), wbig_ref[...],
                 preferred_element_type=_f32)
    for i in range(1, NM + 1):
        mi = a[:, CH * i:CH * (i + 1)]
        mw = jnp.dot(mi, wa_ref[...], preferred_element_type=_f32) + ba_ref[...]
        ci = cc[:, D * (i - 1):D * i] + mb_ref[i - 1]
        att = 1.0 / (1.0 + jnp.exp(-jnp.sum(mw * ci, axis=1, keepdims=True)))
        o_ref[:, D * (i - 1):D * i] = att * (mw - ci)


def _attention(h, mot, wbig, wa, ba, mb):
    bn = 1000
    return pl.pallas_call(
        _attn_body,
        grid=(N // bn,),
        in_specs=[
            pl.BlockSpec((bn, CH), lambda i: (i, 0)),
            pl.BlockSpec((NM + 1, bn, CH), lambda i: (0, i, 0)),
            pl.BlockSpec(((NM + 1) * CH, NM * D), lambda i: (0, 0)),
            pl.BlockSpec((CH, D), lambda i: (0, 0)),
            pl.BlockSpec((D,), lambda i: (0,)),
            pl.BlockSpec((NM, D), lambda i: (0, 0)),
        ],
        out_specs=pl.BlockSpec((bn, NM * D), lambda i: (i, 0)),
        out_shape=jax.ShapeDtypeStruct((N, NM * D), _f32),
    )(h, mot, wbig, wa, ba, mb)


def kernel(x, edge_src, edge_dst, edge_w, motif_row, motif_col, motif_val,
           weight, root, bias, wa, ba, motif_weights, motif_biases):
    padE = EPAD - E
    tailB = ROWS_B * B - R * EPAD
    r_off = (jnp.arange(R, dtype=_i32) * NPAD)[:, None]
    spread = (jnp.arange(padE, dtype=_i32) * 157) % N
    colb = jnp.concatenate(
        [jnp.concatenate([edge_src + r_off,
                          jnp.broadcast_to(spread, (R, padE))],
                         axis=1).reshape(-1),
         (jnp.arange(tailB, dtype=_i32) * 157) % N]).reshape(ROWS_B, B)
    rowb = jnp.concatenate(
        [jnp.concatenate([edge_dst, jnp.full((R, padE), DUMP, _i32)],
                         axis=1).reshape(-1),
         jnp.full((tailB,), DUMP, _i32)]).reshape(ROWS_B, B)
    valb = jnp.concatenate(
        [jnp.concatenate([edge_w, jnp.zeros((R, padE), _f32)],
                         axis=1).reshape(-1),
         jnp.zeros((tailB,), _f32)]).reshape(ROWS_B, B)

    padM = EPAD_M - M
    spreadM = (jnp.arange(padM, dtype=_i32) * 157) % N
    colm = jnp.concatenate(
        [motif_col, jnp.broadcast_to(spreadM, (NM, padM))],
        axis=1).reshape(NM * ROWS_M1, B)
    rowm = jnp.concatenate(
        [motif_row, jnp.full((NM, padM), DUMP, _i32)],
        axis=1).reshape(NM * ROWS_M1, B)
    valm = jnp.concatenate(
        [motif_val, jnp.zeros((NM, padM), _f32)],
        axis=1).reshape(NM * ROWS_M1, B)

    wbig_cols = []
    zero_blk = jnp.zeros((CH, D), _f32)
    for i in range(1, NM + 1):
        wi = motif_weights[i - 1]
        wbig_cols.append(jnp.concatenate(
            [wi[:CH * i], zero_blk, wi[CH * i:]], axis=0))
    wbig = jnp.concatenate(wbig_cols, axis=1).astype(jnp.bfloat16)

    w4 = jnp.concatenate([weight, root[None]], axis=0)
    x_pad = jnp.concatenate([x, jnp.zeros((NPAD - N, CH), _f32)], axis=0)

    xt4 = _compute_xt(x_pad, w4)
    xt_flat = xt4.reshape(4 * NPAD, CH)
    hp, degp = _relation_aggregate(xt_flat, colb, rowb, valb)
    h = _combine(hp, degp, xt4, bias)
    mot = _motif_spmm(h, colm, rowm, valm)
    return _attention(h, mot, wbig, wa, ba, motif_biases)
